# baseline (speedup 1.0000x reference)
import functools

import jax
import jax.numpy as jnp
from jax import lax
from jax.experimental import pallas as pl
from jax.experimental.pallas import tpu as pltpu
from jax.experimental.pallas import tpu_sc as plsc

_MAX_LEN = 8192
_D_MODEL = 768
_NUM_WORKERS = 32
_ROWS_PER_WORKER = _MAX_LEN // _NUM_WORKERS


_CHUNK_ROWS = 32
_NBUF = 4
_CHUNK_SIZES = (16, 16) + (32,) * 7
_NUM_CHUNKS = len(_CHUNK_SIZES)
_CHUNK_OFFS = tuple(sum(_CHUNK_SIZES[:j]) for j in range(_NUM_CHUNKS))


@functools.partial(
    pl.kernel,
    out_type=jax.ShapeDtypeStruct((_MAX_LEN, _D_MODEL), jnp.float32),
    mesh=plsc.VectorSubcoreMesh(core_axis_name="c", subcore_axis_name="s"),
)
def _pos_encoding_copy(emb_hbm, out_hbm):
    pl.run_scoped(
        functools.partial(_worker_body, emb_hbm, out_hbm),
        pltpu.VMEM((_NBUF, _CHUNK_ROWS, _D_MODEL), jnp.float32),
        pltpu.SemaphoreType.DMA,
        pltpu.SemaphoreType.DMA,
        pltpu.SemaphoreType.DMA,
        pltpu.SemaphoreType.DMA,
    )


def _worker_body(emb_hbm, out_hbm, buf, in_sem0, in_sem1, out_sem0, out_sem1):
    wid = lax.axis_index("s") * 2 + lax.axis_index("c")
    base = wid * _ROWS_PER_WORKER

    def copy_in(j):
        n = _CHUNK_SIZES[j]
        return pltpu.async_copy(
            emb_hbm.at[pl.ds(base + _CHUNK_OFFS[j], n)],
            buf.at[j % _NBUF, pl.ds(0, n)],
            in_sem0 if j % 2 == 0 else in_sem1,
        )

    def copy_out(j):
        n = _CHUNK_SIZES[j]
        return pltpu.async_copy(
            buf.at[j % _NBUF, pl.ds(0, n)],
            out_hbm.at[pl.ds(base + _CHUNK_OFFS[j], n)],
            out_sem0 if j % 2 == 0 else out_sem1,
        )

    ins = [None] * _NUM_CHUNKS
    outs = [None] * _NUM_CHUNKS
    for j in range(_NBUF):
        ins[j] = copy_in(j)
    for j in range(_NUM_CHUNKS):
        ins[j].wait()
        outs[j] = copy_out(j)
        nxt = j + _NBUF
        if nxt < _NUM_CHUNKS:
            outs[j].wait()
            ins[nxt] = copy_in(nxt)
    for j in range(_NUM_CHUNKS - _NBUF, _NUM_CHUNKS):
        if j >= 0:
            outs[j].wait()


def kernel(x, embedding):
    del x
    return _pos_encoding_copy(embedding)[None]


# speedup vs baseline: 1.0045x; 1.0045x over previous
import functools

import jax
import jax.numpy as jnp
from jax import lax
from jax.experimental import pallas as pl
from jax.experimental.pallas import tpu as pltpu
from jax.experimental.pallas import tpu_sc as plsc

_MAX_LEN = 8192
_D_MODEL = 768
_NUM_WORKERS = 32
_ROWS_PER_WORKER = _MAX_LEN // _NUM_WORKERS


_CHUNK_ROWS = 32
_NUM_CHUNKS = _ROWS_PER_WORKER // _CHUNK_ROWS
_NBUF = 4
_NQ = 3


@functools.partial(
    pl.kernel,
    out_type=jax.ShapeDtypeStruct((_MAX_LEN, _D_MODEL), jnp.float32),
    mesh=plsc.VectorSubcoreMesh(core_axis_name="c", subcore_axis_name="s"),
)
def _pos_encoding_copy(emb_hbm, out_hbm):
    pl.run_scoped(
        functools.partial(_worker_body, emb_hbm, out_hbm),
        pltpu.VMEM((_NBUF, _CHUNK_ROWS, _D_MODEL), jnp.float32),
        *([pltpu.SemaphoreType.DMA] * (2 * _NQ)),
    )


def _worker_body(emb_hbm, out_hbm, buf, *sems):
    in_sems, out_sems = sems[:_NQ], sems[_NQ:]
    wid = lax.axis_index("s") * 2 + lax.axis_index("c")
    base = wid * _ROWS_PER_WORKER

    def copy_in(j):
        return pltpu.async_copy(
            emb_hbm.at[pl.ds(base + j * _CHUNK_ROWS, _CHUNK_ROWS)],
            buf.at[j % _NBUF],
            in_sems[j % _NQ],
        )

    def copy_out(j):
        return pltpu.async_copy(
            buf.at[j % _NBUF],
            out_hbm.at[pl.ds(base + j * _CHUNK_ROWS, _CHUNK_ROWS)],
            out_sems[j % _NQ],
        )

    ins = [None] * _NUM_CHUNKS
    outs = [None] * _NUM_CHUNKS
    for j in range(_NBUF):
        ins[j] = copy_in(j)
    for j in range(_NUM_CHUNKS):
        ins[j].wait()
        outs[j] = copy_out(j)
        nxt = j + _NBUF
        if nxt < _NUM_CHUNKS:
            outs[j].wait()
            ins[nxt] = copy_in(nxt)
    for j in range(_NUM_CHUNKS - _NBUF, _NUM_CHUNKS):
        if j >= 0:
            outs[j].wait()


def kernel(x, embedding):
    del x
    return _pos_encoding_copy(embedding)[None]


# speedup vs baseline: 1.0132x; 1.0087x over previous
import functools

import jax
import jax.numpy as jnp
from jax import lax
from jax.experimental import pallas as pl
from jax.experimental.pallas import tpu as pltpu
from jax.experimental.pallas import tpu_sc as plsc

_MAX_LEN = 8192
_D_MODEL = 768
_NUM_WORKERS = 32
_ROWS_PER_WORKER = _MAX_LEN // _NUM_WORKERS


_CHUNK_ROWS = 32
_NUM_CHUNKS = _ROWS_PER_WORKER // _CHUNK_ROWS
_NBUF = 4
_NQ = 2


@functools.partial(
    pl.kernel,
    out_type=jax.ShapeDtypeStruct((_MAX_LEN, _D_MODEL), jnp.float32),
    mesh=plsc.VectorSubcoreMesh(core_axis_name="c", subcore_axis_name="s"),
)
def _pos_encoding_copy(emb_hbm, out_hbm):
    pl.run_scoped(
        functools.partial(_worker_body, emb_hbm, out_hbm),
        pltpu.VMEM((_NBUF, _CHUNK_ROWS, _D_MODEL), jnp.float32),
        *([pltpu.SemaphoreType.DMA] * (2 * _NQ)),
    )


def _worker_body(emb_hbm, out_hbm, buf, *sems):
    in_sems, out_sems = sems[:_NQ], sems[_NQ:]
    wid = lax.axis_index("c") * _NUM_WORKERS // 2 + lax.axis_index("s")
    base = wid * _ROWS_PER_WORKER

    def copy_in(j):
        return pltpu.async_copy(
            emb_hbm.at[pl.ds(base + j * _CHUNK_ROWS, _CHUNK_ROWS)],
            buf.at[j % _NBUF],
            in_sems[j % _NQ],
        )

    def copy_out(j):
        return pltpu.async_copy(
            buf.at[j % _NBUF],
            out_hbm.at[pl.ds(base + j * _CHUNK_ROWS, _CHUNK_ROWS)],
            out_sems[j % _NQ],
        )

    ins = [None] * _NUM_CHUNKS
    outs = [None] * _NUM_CHUNKS
    for j in range(_NBUF):
        ins[j] = copy_in(j)
    for j in range(_NUM_CHUNKS):
        ins[j].wait()
        outs[j] = copy_out(j)
        nxt = j + _NBUF
        if nxt < _NUM_CHUNKS:
            outs[j].wait()
            ins[nxt] = copy_in(nxt)
    for j in range(_NUM_CHUNKS - _NBUF, _NUM_CHUNKS):
        if j >= 0:
            outs[j].wait()


def kernel(x, embedding):
    del x
    return _pos_encoding_copy(embedding)[None]
